# Initial kernel scaffold; baseline (speedup 1.0000x reference)
import jax
import jax.numpy as jnp
from jax.experimental import pallas as pl


def kernel(x, edge_index, edge_feat, l0_eW1, l0_eb1, l0_eW2, l0_eb2, l0_nW1, l0_nb1, l0_nW2, l0_nb2, l1_eW1, l1_eb1, l1_eW2, l1_eb2, l1_nW1, l1_nb1, l1_nW2, l1_nb2):
    raise NotImplementedError("write your pallas kernel here")


# speedup vs baseline: 1.2117x; 1.2117x over previous
import functools
import jax
import jax.numpy as jnp
from jax import lax
from jax.experimental import pallas as pl
from jax.experimental.pallas import tpu as pltpu
from jax.experimental.pallas import tpu_sc as plsc

N = 10000
E = 320000
D = 128
ED = 16
TW = 160
PW = 144
NC = 2
NS = 16
NW = NC * NS
CH = 128
NCHUNK = E // CH
NB = 1000
EB = 2000
ZR = 125

_f32 = jnp.float32


def _tcA_body(h_ref, wd_ref, ws_ref, bd_ref, td_ref, ts_ref):
    h = h_ref[...]
    td_ref[...] = jnp.dot(h, wd_ref[...], preferred_element_type=_f32) + bd_ref[...]
    ts_ref[...] = jnp.dot(h, ws_ref[...], preferred_element_type=_f32)


def _tcA(h, Wd, Ws, bd):
    grid = N // NB
    return pl.pallas_call(
        _tcA_body,
        grid=(grid,),
        in_specs=[
            pl.BlockSpec((NB, D), lambda i: (i, 0)),
            pl.BlockSpec((D, TW), lambda i: (0, 0)),
            pl.BlockSpec((D, TW), lambda i: (0, 0)),
            pl.BlockSpec((1, TW), lambda i: (0, 0)),
        ],
        out_specs=[
            pl.BlockSpec((NB, TW), lambda i: (i, 0)),
            pl.BlockSpec((NB, TW), lambda i: (i, 0)),
        ],
        out_shape=[
            jax.ShapeDtypeStruct((N, TW), _f32),
            jax.ShapeDtypeStruct((N, TW), _f32),
        ],
    )(h, Wd, Ws, bd)


def _tcB_body(gd_ref, gs_ref, ef_ref, ew1c_ref, ew2_ref, eb2_ref, nw1c_ref,
              ef2_ref, u_ref):
    s = gd_ref[...] + gs_ref[...]
    t = jnp.maximum(
        s[:, :32] + jnp.dot(ef_ref[...], ew1c_ref[...], preferred_element_type=_f32),
        0.0)
    ef2 = jnp.dot(t, ew2_ref[...], preferred_element_type=_f32) + eb2_ref[...]
    ef2_ref[...] = ef2
    u = jnp.maximum(
        s[:, 32:] + jnp.dot(ef2, nw1c_ref[...], preferred_element_type=_f32),
        0.0)
    ones = jnp.ones((u.shape[0], 1), _f32)
    zeros = jnp.zeros((u.shape[0], PW - D - 1), _f32)
    u_ref[...] = jnp.concatenate([u, ones, zeros], axis=1)


def _tcB(gd, gs, ef, eW1c, eW2, eb2, nW1c):
    grid = E // EB
    return pl.pallas_call(
        _tcB_body,
        grid=(grid,),
        in_specs=[
            pl.BlockSpec((EB, TW), lambda i: (i, 0)),
            pl.BlockSpec((EB, TW), lambda i: (i, 0)),
            pl.BlockSpec((EB, ED), lambda i: (i, 0)),
            pl.BlockSpec((ED, 32), lambda i: (0, 0)),
            pl.BlockSpec((32, ED), lambda i: (0, 0)),
            pl.BlockSpec((1, ED), lambda i: (0, 0)),
            pl.BlockSpec((ED, D), lambda i: (0, 0)),
        ],
        out_specs=[
            pl.BlockSpec((EB, ED), lambda i: (i, 0)),
            pl.BlockSpec((EB, PW), lambda i: (i, 0)),
        ],
        out_shape=[
            jax.ShapeDtypeStruct((E, ED), _f32),
            jax.ShapeDtypeStruct((E, PW), _f32),
        ],
    )(gd, gs, ef, eW1c, eW2, eb2, nW1c)


def _tcC_body(agg_ref, w_ref, h_ref):
    a = agg_ref[0] + agg_ref[1]
    h_ref[...] = jnp.dot(a, w_ref[...], preferred_element_type=_f32)


def _tcC(agg, W144):
    grid = N // NB
    return pl.pallas_call(
        _tcC_body,
        grid=(grid,),
        in_specs=[
            pl.BlockSpec((2, NB, PW), lambda i: (0, i, 0)),
            pl.BlockSpec((PW, D), lambda i: (0, 0)),
        ],
        out_specs=pl.BlockSpec((NB, D), lambda i: (i, 0)),
        out_shape=jax.ShapeDtypeStruct((N, D), _f32),
    )(agg, W144)


def _mesh():
    return plsc.VectorSubcoreMesh(core_axis_name="c", subcore_axis_name="s",
                                  num_cores=NC, num_subcores=NS)


def _sc_gather_body(td_hbm, ts_hbm, ei_hbm, gd_hbm, gs_hbm,
                    idxd_v, idxs_v, gd_v, gs_v, sem1, sem2):
    cid = lax.axis_index("c")
    sid = lax.axis_index("s")
    wid = sid * NC + cid
    nk = (NCHUNK - wid + NW - 1) // NW

    def body(k, _):
        off = (wid + k * NW) * CH
        pltpu.sync_copy(ei_hbm.at[1, pl.ds(off, CH)], idxd_v)
        pltpu.sync_copy(ei_hbm.at[0, pl.ds(off, CH)], idxs_v)
        cp1 = pltpu.async_copy(td_hbm.at[idxd_v], gd_v, sem1)
        cp2 = pltpu.async_copy(ts_hbm.at[idxs_v], gs_v, sem2)
        cp1.wait()
        cp2.wait()
        pltpu.sync_copy(gd_v, gd_hbm.at[pl.ds(off, CH)])
        pltpu.sync_copy(gs_v, gs_hbm.at[pl.ds(off, CH)])
        return _

    lax.fori_loop(0, nk, body, None)


def _sc_gather(td, ts, edge_index):
    fn = pl.kernel(
        _sc_gather_body,
        out_type=[
            jax.ShapeDtypeStruct((E, TW), _f32),
            jax.ShapeDtypeStruct((E, TW), _f32),
        ],
        mesh=_mesh(),
        scratch_types=[
            pltpu.VMEM((CH,), jnp.int32),
            pltpu.VMEM((CH,), jnp.int32),
            pltpu.VMEM((CH, TW), _f32),
            pltpu.VMEM((CH, TW), _f32),
            pltpu.SemaphoreType.DMA,
            pltpu.SemaphoreType.DMA,
        ],
        compiler_params=pltpu.CompilerParams(use_tc_tiling_on_sc=False),
    )
    return fn(td, ts, edge_index)


def _sc_scatter_body(u_hbm, ei_hbm, out_hbm, idx_v, pay_v, zer_v, acc_sh, sem):
    cid = lax.axis_index("c")
    sid = lax.axis_index("s")
    wid = sid * NC + cid

    zvec = jnp.zeros((16,), _f32)

    def zbody(i, _):
        r = i // (PW // 16)
        c = (i % (PW // 16)) * 16
        zer_v[r, pl.ds(c, 16)] = zvec
        return _

    lax.fori_loop(0, ZR * (PW // 16), zbody, None)
    for j in range(5):
        pltpu.sync_copy(zer_v, acc_sh.at[pl.ds(sid * 625 + j * ZR, ZR)])
    plsc.subcore_barrier()

    nk = (NCHUNK - wid + NW - 1) // NW

    def body(k, _):
        off = (wid + k * NW) * CH
        pltpu.sync_copy(ei_hbm.at[1, pl.ds(off, CH)], idx_v)
        pltpu.sync_copy(u_hbm.at[pl.ds(off, CH)], pay_v)
        pltpu.sync_copy(pay_v, acc_sh.at[idx_v], add=True)
        return _

    lax.fori_loop(0, nk, body, None)
    plsc.subcore_barrier()
    pltpu.sync_copy(acc_sh.at[pl.ds(sid * 625, 625)],
                    out_hbm.at[cid, pl.ds(sid * 625, 625)])


def _sc_scatter(u, edge_index):
    fn = pl.kernel(
        _sc_scatter_body,
        out_type=jax.ShapeDtypeStruct((NC, N, PW), _f32),
        mesh=_mesh(),
        scratch_types=[
            pltpu.VMEM((CH,), jnp.int32),
            pltpu.VMEM((CH, PW), _f32),
            pltpu.VMEM((ZR, PW), _f32),
            pltpu.VMEM_SHARED((N, PW), _f32),
            pltpu.SemaphoreType.DMA,
        ],
        compiler_params=pltpu.CompilerParams(use_tc_tiling_on_sc=False),
    )
    return fn(u, edge_index)


def _layer(h, ef, edge_index, eW1, eb1, eW2, eb2, nW1, nb1, nW2, nb2):
    Wd = jnp.concatenate([eW1[:D], nW1[:D]], axis=1)
    Ws = jnp.concatenate([eW1[D:2 * D], nW1[D:2 * D]], axis=1)
    bd = jnp.concatenate([eb1, nb1])[None, :]
    eW1c = eW1[2 * D:]
    nW1c = nW1[2 * D:]
    W144 = jnp.concatenate(
        [nW2, nb2[None, :], jnp.zeros((PW - D - 1, D), _f32)], axis=0)

    td, ts = _tcA(h, Wd, Ws, bd)
    gd, gs = _sc_gather(td, ts, edge_index)
    ef2, u = _tcB(gd, gs, ef, eW1c, eW2, eb2[None, :], nW1c)
    agg = _sc_scatter(u, edge_index)
    h2 = _tcC(agg, W144)
    return h2, ef2


def kernel(x, edge_index, edge_feat,
           l0_eW1, l0_eb1, l0_eW2, l0_eb2, l0_nW1, l0_nb1, l0_nW2, l0_nb2,
           l1_eW1, l1_eb1, l1_eW2, l1_eb2, l1_nW1, l1_nb1, l1_nW2, l1_nb2):
    h1, ef1 = _layer(x, edge_feat, edge_index,
                     l0_eW1, l0_eb1, l0_eW2, l0_eb2,
                     l0_nW1, l0_nb1, l0_nW2, l0_nb2)
    h2, ef2 = _layer(h1, ef1, edge_index,
                     l1_eW1, l1_eb1, l1_eW2, l1_eb2,
                     l1_nW1, l1_nb1, l1_nW2, l1_nb2)
    return (h2, ef2)


# speedup vs baseline: 2.7127x; 2.2387x over previous
import jax
import jax.numpy as jnp
from jax import lax
from jax.experimental import pallas as pl
from jax.experimental.pallas import tpu as pltpu
from jax.experimental.pallas import tpu_sc as plsc

N = 10000
E = 320000
D = 128
ED = 16
NC = 2
NS = 16
NW = NC * NS
CH = 128
NCHUNK = E // CH
NB = 1000
EB = 4000
ZR = 128
NP = 10240

_f32 = jnp.float32


def _tcB_body(xi_ref, xj_ref, ef_ref,
              ew1a_ref, ew1b_ref, ew1c_ref, eb1_ref, ew2_ref, eb2_ref,
              nw1a_ref, nw1b_ref, nw1c_ref, nb1_ref,
              ef2_ref, u_ref):
    xi = xi_ref[...]
    xj = xj_ref[...]
    ef = ef_ref[...]
    pe = (jnp.dot(xi, ew1a_ref[...], preferred_element_type=_f32)
          + jnp.dot(xj, ew1b_ref[...], preferred_element_type=_f32)
          + jnp.dot(ef, ew1c_ref[...], preferred_element_type=_f32)
          + eb1_ref[...])
    t = jnp.maximum(pe, 0.0)
    ef2 = jnp.dot(t, ew2_ref[...], preferred_element_type=_f32) + eb2_ref[...]
    ef2_ref[...] = ef2
    pn = (jnp.dot(xi, nw1a_ref[...], preferred_element_type=_f32)
          + jnp.dot(xj, nw1b_ref[...], preferred_element_type=_f32)
          + jnp.dot(ef2, nw1c_ref[...], preferred_element_type=_f32)
          + nb1_ref[...])
    u_ref[...] = jnp.maximum(pn, 0.0)


def _tcB(xi, xj, ef, eW1a, eW1b, eW1c, eb1, eW2, eb2, nW1a, nW1b, nW1c, nb1):
    grid = E // EB
    full = lambda shape: pl.BlockSpec(shape, lambda i: tuple(0 for _ in shape))
    return pl.pallas_call(
        _tcB_body,
        grid=(grid,),
        in_specs=[
            pl.BlockSpec((EB, D), lambda i: (i, 0)),
            pl.BlockSpec((EB, D), lambda i: (i, 0)),
            pl.BlockSpec((EB, ED), lambda i: (i, 0)),
            full((D, 32)), full((D, 32)), full((ED, 32)), full((1, 32)),
            full((32, ED)), full((1, ED)),
            full((D, D)), full((D, D)), full((ED, D)), full((1, D)),
        ],
        out_specs=[
            pl.BlockSpec((EB, ED), lambda i: (i, 0)),
            pl.BlockSpec((EB, D), lambda i: (i, 0)),
        ],
        out_shape=[
            jax.ShapeDtypeStruct((E, ED), _f32),
            jax.ShapeDtypeStruct((E, D), _f32),
        ],
    )(xi, xj, ef, eW1a, eW1b, eW1c, eb1, eW2, eb2, nW1a, nW1b, nW1c, nb1)


def _tcC_body(agg_ref, deg_ref, w_ref, b_ref, h_ref):
    a = agg_ref[0] + agg_ref[1]
    d = deg_ref[0, :, 0:1] + deg_ref[1, :, 0:1]
    h_ref[...] = jnp.dot(a, w_ref[...], preferred_element_type=_f32) + d * b_ref[...]


def _tcC(agg, deg, nW2, nb2):
    grid = N // NB
    return pl.pallas_call(
        _tcC_body,
        grid=(grid,),
        in_specs=[
            pl.BlockSpec((2, NB, D), lambda i: (0, i, 0)),
            pl.BlockSpec((2, NB, ED), lambda i: (0, i, 0)),
            pl.BlockSpec((D, D), lambda i: (0, 0)),
            pl.BlockSpec((1, D), lambda i: (0, 0)),
        ],
        out_specs=pl.BlockSpec((NB, D), lambda i: (i, 0)),
        out_shape=jax.ShapeDtypeStruct((N, D), _f32),
    )(agg, deg, nW2, nb2)


def _mesh():
    return plsc.VectorSubcoreMesh(core_axis_name="c", subcore_axis_name="s",
                                  num_cores=NC, num_subcores=NS)


def _sc_gather_body(h_hbm, dst_hbm, src_hbm, xi_hbm, xj_hbm,
                    idxd_v, idxs_v, xi_v, xj_v, sem1, sem2):
    cid = lax.axis_index("c")
    sid = lax.axis_index("s")
    wid = sid * NC + cid
    nk = (NCHUNK - wid + NW - 1) // NW

    def body(k, carry):
        off = (wid + k * NW) * CH
        pltpu.sync_copy(dst_hbm.at[pl.ds(off, CH)], idxd_v)
        pltpu.sync_copy(src_hbm.at[pl.ds(off, CH)], idxs_v)
        cp1 = pltpu.async_copy(h_hbm.at[idxd_v], xi_v, sem1)
        cp2 = pltpu.async_copy(h_hbm.at[idxs_v], xj_v, sem2)
        cp1.wait()
        cp2.wait()
        pltpu.sync_copy(xi_v, xi_hbm.at[pl.ds(off, CH)])
        pltpu.sync_copy(xj_v, xj_hbm.at[pl.ds(off, CH)])
        return carry

    lax.fori_loop(0, nk, body, None)


def _sc_gather(h, dst, src):
    fn = pl.kernel(
        _sc_gather_body,
        out_type=[
            jax.ShapeDtypeStruct((E, D), _f32),
            jax.ShapeDtypeStruct((E, D), _f32),
        ],
        mesh=_mesh(),
        scratch_types=[
            pltpu.VMEM((CH,), jnp.int32),
            pltpu.VMEM((CH,), jnp.int32),
            pltpu.VMEM((CH, D), _f32),
            pltpu.VMEM((CH, D), _f32),
            pltpu.SemaphoreType.DMA,
            pltpu.SemaphoreType.DMA,
        ],
    )
    return fn(h, dst, src)


def _sc_scatter_body(u_hbm, dst_hbm, out_hbm, idx_v, pay_v, zer_v, acc_sh, sem):
    cid = lax.axis_index("c")
    sid = lax.axis_index("s")
    wid = sid * NC + cid

    zvec = jnp.zeros((16,), _f32)

    def zbody(i, carry):
        r = i // (D // 16)
        c = (i % (D // 16)) * 16
        zer_v[r, pl.ds(c, 16)] = zvec
        return carry

    lax.fori_loop(0, ZR * (D // 16), zbody, None)
    for j in range(5):
        pltpu.sync_copy(zer_v, acc_sh.at[pl.ds(sid * 640 + j * ZR, ZR)])
    plsc.subcore_barrier()

    nk = (NCHUNK - wid + NW - 1) // NW

    def body(k, carry):
        off = (wid + k * NW) * CH
        pltpu.sync_copy(dst_hbm.at[pl.ds(off, CH)], idx_v)
        pltpu.sync_copy(u_hbm.at[pl.ds(off, CH)], pay_v)
        pltpu.sync_copy(pay_v, acc_sh.at[idx_v], add=True)
        return carry

    lax.fori_loop(0, nk, body, None)
    plsc.subcore_barrier()
    pltpu.sync_copy(acc_sh.at[pl.ds(sid * 640, 640)],
                    out_hbm.at[cid, pl.ds(sid * 640, 640)])


def _sc_scatter(u, dst):
    fn = pl.kernel(
        _sc_scatter_body,
        out_type=jax.ShapeDtypeStruct((NC, NP, D), _f32),
        mesh=_mesh(),
        scratch_types=[
            pltpu.VMEM((CH,), jnp.int32),
            pltpu.VMEM((CH, D), _f32),
            pltpu.VMEM((ZR, D), _f32),
            pltpu.VMEM_SHARED((NP, D), _f32),
            pltpu.SemaphoreType.DMA,
        ],
    )
    return fn(u, dst)


def _sc_deg_body(dst_hbm, out_hbm, idx_v, one_v, zer_v, acc_sh, sem):
    cid = lax.axis_index("c")
    sid = lax.axis_index("s")
    wid = sid * NC + cid

    onev = jnp.ones((16,), _f32)
    zvec = jnp.zeros((16,), _f32)

    def obody(i, carry):
        one_v[i, pl.ds(0, 16)] = onev
        return carry

    lax.fori_loop(0, CH, obody, None)

    def zbody(i, carry):
        zer_v[i, pl.ds(0, 16)] = zvec
        return carry

    lax.fori_loop(0, ZR, zbody, None)
    for j in range(5):
        pltpu.sync_copy(zer_v, acc_sh.at[pl.ds(sid * 625 + j * ZR, ZR)])
    plsc.subcore_barrier()

    nk = (NCHUNK - wid + NW - 1) // NW

    def body(k, carry):
        off = (wid + k * NW) * CH
        pltpu.sync_copy(dst_hbm.at[pl.ds(off, CH)], idx_v)
        pltpu.sync_copy(one_v, acc_sh.at[idx_v], add=True)
        return carry

    lax.fori_loop(0, nk, body, None)
    plsc.subcore_barrier()
    pltpu.sync_copy(acc_sh.at[pl.ds(sid * 625, 625)],
                    out_hbm.at[cid, pl.ds(sid * 625, 625)])


def _sc_deg(dst):
    fn = pl.kernel(
        _sc_deg_body,
        out_type=jax.ShapeDtypeStruct((NC, N, ED), _f32),
        mesh=_mesh(),
        scratch_types=[
            pltpu.VMEM((CH,), jnp.int32),
            pltpu.VMEM((CH, ED), _f32),
            pltpu.VMEM((ZR, ED), _f32),
            pltpu.VMEM_SHARED((N, ED), _f32),
            pltpu.SemaphoreType.DMA,
        ],
        compiler_params=pltpu.CompilerParams(use_tc_tiling_on_sc=False),
    )
    return fn(dst)


def _layer(h, ef, dst, src, deg,
           eW1, eb1, eW2, eb2, nW1, nb1, nW2, nb2):
    xi, xj = _sc_gather(h, dst, src)
    ef2, u = _tcB(xi, xj, ef,
                  eW1[:D], eW1[D:2 * D], eW1[2 * D:], eb1[None, :],
                  eW2, eb2[None, :],
                  nW1[:D], nW1[D:2 * D], nW1[2 * D:], nb1[None, :])
    agg = _sc_scatter(u, dst)
    h2 = _tcC(agg, deg, nW2, nb2[None, :])
    return h2, ef2


def kernel(x, edge_index, edge_feat,
           l0_eW1, l0_eb1, l0_eW2, l0_eb2, l0_nW1, l0_nb1, l0_nW2, l0_nb2,
           l1_eW1, l1_eb1, l1_eW2, l1_eb2, l1_nW1, l1_nb1, l1_nW2, l1_nb2):
    src = edge_index[0]
    dst = edge_index[1]
    deg = _sc_deg(dst)
    h1, ef1 = _layer(x, edge_feat, dst, src, deg,
                     l0_eW1, l0_eb1, l0_eW2, l0_eb2,
                     l0_nW1, l0_nb1, l0_nW2, l0_nb2)
    h2, ef2 = _layer(h1, ef1, dst, src, deg,
                     l1_eW1, l1_eb1, l1_eW2, l1_eb2,
                     l1_nW1, l1_nb1, l1_nW2, l1_nb2)
    return (h2, ef2)
